# baseline (speedup 1.0000x reference)
import jax
import jax.numpy as jnp
from jax import lax
from jax.experimental import pallas as pl
from jax.experimental.pallas import tpu as pltpu
from jax.experimental.pallas import tpu_sc as plsc

D = 64
W = 200
B = 4096
NC, NS = 2, 16
NW = NC * NS
TOTAL = B * W
BPW = TOTAL // NW
CHUNK = W
NCHUNK = BPW // CHUNK
SUB_A, SUB_B = 120, 80
LANES = 16


def _embed_body(vec_hbm, table_hbm, pe_hbm, out_hbm,
                idx_a, idx_b, rows_a, rows_b, pe_v, sem_a, sem_b):
    wid = lax.axis_index("s") * NC + lax.axis_index("c")
    base0 = wid * BPW
    pltpu.sync_copy(pe_hbm, pe_v)

    def chunk_body(g, carry):
        base = base0 + g * CHUNK
        pltpu.sync_copy(vec_hbm.at[pl.ds(base, SUB_A)], idx_a)
        pltpu.sync_copy(vec_hbm.at[pl.ds(base + SUB_A, SUB_B)], idx_b)
        cpa = pltpu.async_copy(table_hbm.at[idx_a], rows_a, sem_a)
        cpb = pltpu.async_copy(table_hbm.at[idx_b], rows_b, sem_b)
        cpa.wait()
        cpb.wait()

        def add_a(i, c):
            for j in range(D // LANES):
                s = pl.ds(j * LANES, LANES)
                rows_a[i, s] = rows_a[i, s] + pe_v[i, s]
            return c

        lax.fori_loop(0, SUB_A, add_a, 0)

        def add_b(i, c):
            for j in range(D // LANES):
                s = pl.ds(j * LANES, LANES)
                rows_b[i, s] = rows_b[i, s] + pe_v[SUB_A + i, s]
            return c

        lax.fori_loop(0, SUB_B, add_b, 0)

        pltpu.sync_copy(rows_a, out_hbm.at[pl.ds(base, SUB_A)])
        pltpu.sync_copy(rows_b, out_hbm.at[pl.ds(base + SUB_A, SUB_B)])
        return carry

    lax.fori_loop(0, NCHUNK, chunk_body, 0)


_embed_call = pl.kernel(
    _embed_body,
    mesh=plsc.VectorSubcoreMesh(core_axis_name="c", subcore_axis_name="s"),
    out_type=jax.ShapeDtypeStruct((TOTAL, D), jnp.float32),
    scratch_types=[
        pltpu.VMEM((SUB_A,), jnp.int32),
        pltpu.VMEM((SUB_B,), jnp.int32),
        pltpu.VMEM((SUB_A, D), jnp.float32),
        pltpu.VMEM((SUB_B, D), jnp.float32),
        pltpu.VMEM((W, D), jnp.float32),
        pltpu.SemaphoreType.DMA,
        pltpu.SemaphoreType.DMA,
    ],
    compiler_params=pltpu.CompilerParams(use_tc_tiling_on_sc=False),
)


def kernel(vector, table, pe):
    vec_flat = vector.reshape(TOTAL)
    out = _embed_call(vec_flat, table, pe)
    return out.reshape(B, W, D)


# speedup vs baseline: 1.2452x; 1.2452x over previous
import jax
import jax.numpy as jnp
from jax import lax
from jax.experimental import pallas as pl
from jax.experimental.pallas import tpu as pltpu
from jax.experimental.pallas import tpu_sc as plsc

D = 64
W = 200
B = 4096
NC, NS = 2, 16
NW = NC * NS
TOTAL = B * W
BPW = TOTAL // NW
CHUNK = W
NCHUNK = BPW // CHUNK
SUB_A, SUB_B = 120, 80
LANES = 16
NBUF = 4
PF = 2


def _embed_body(idxa_hbm, idxb_hbm, table_hbm, pe_hbm, out_hbm,
                idx_all_a, idx_all_b, pe_v,
                ra0, ra1, ra2, ra3, rb0, rb1, rb2, rb3,
                sga0, sga1, sga2, sga3, sgb0, sgb1, sgb2, sgb3,
                ssa0, ssa1, ssa2, ssa3, ssb0, ssb1, ssb2, ssb3):
    ra = [ra0, ra1, ra2, ra3]
    rb = [rb0, rb1, rb2, rb3]
    sga = [sga0, sga1, sga2, sga3]
    sgb = [sgb0, sgb1, sgb2, sgb3]
    ssa = [ssa0, ssa1, ssa2, ssa3]
    ssb = [ssb0, ssb1, ssb2, ssb3]

    wid = lax.axis_index("s") * NC + lax.axis_index("c")
    base0 = wid * BPW
    pltpu.sync_copy(pe_hbm, pe_v)
    pltpu.sync_copy(idxa_hbm.at[wid], idx_all_a)
    pltpu.sync_copy(idxb_hbm.at[wid], idx_all_b)

    def start_gather(g, s):
        pltpu.async_copy(table_hbm.at[idx_all_a.at[g]], ra[s], sga[s])
        pltpu.async_copy(table_hbm.at[idx_all_b.at[g]], rb[s], sgb[s])

    def wait_gather(g, s):
        pltpu.make_async_copy(table_hbm.at[idx_all_a.at[g]], ra[s], sga[s]).wait()
        pltpu.make_async_copy(table_hbm.at[idx_all_b.at[g]], rb[s], sgb[s]).wait()

    def start_store(g, s):
        base = base0 + g * CHUNK
        pltpu.async_copy(ra[s], out_hbm.at[pl.ds(base, SUB_A)], ssa[s])
        pltpu.async_copy(rb[s], out_hbm.at[pl.ds(base + SUB_A, SUB_B)], ssb[s])

    def wait_store(g, s):
        base = base0 + g * CHUNK
        pltpu.make_async_copy(ra[s], out_hbm.at[pl.ds(base, SUB_A)], ssa[s]).wait()
        pltpu.make_async_copy(rb[s], out_hbm.at[pl.ds(base + SUB_A, SUB_B)], ssb[s]).wait()

    for s in range(PF):
        start_gather(s, s)

    def outer_body(go, carry):
        for s in range(NBUF):
            g = go * NBUF + s
            sp = (s + PF) % NBUF

            @pl.when(g >= PF)
            def _():
                wait_store(g, sp)

            @pl.when(g + PF < NCHUNK)
            def _():
                start_gather(g + PF, sp)

            wait_gather(g, s)

            def add_a(i, c):
                for j in range(D // LANES):
                    sl = pl.ds(j * LANES, LANES)
                    ra[s][i, sl] = ra[s][i, sl] + pe_v[i, sl]
                return c

            lax.fori_loop(0, SUB_A, add_a, 0)

            def add_b(i, c):
                for j in range(D // LANES):
                    sl = pl.ds(j * LANES, LANES)
                    rb[s][i, sl] = rb[s][i, sl] + pe_v[SUB_A + i, sl]
                return c

            lax.fori_loop(0, SUB_B, add_b, 0)

            start_store(g, s)
        return carry

    lax.fori_loop(0, NCHUNK // NBUF, outer_body, 0)

    for g in range(NCHUNK - NBUF + PF, NCHUNK):
        wait_store(g, g % NBUF)


_embed_call = pl.kernel(
    _embed_body,
    mesh=plsc.VectorSubcoreMesh(core_axis_name="c", subcore_axis_name="s"),
    out_type=jax.ShapeDtypeStruct((TOTAL, D), jnp.float32),
    scratch_types=(
        [
            pltpu.VMEM((NCHUNK, SUB_A), jnp.int32),
            pltpu.VMEM((NCHUNK, SUB_B), jnp.int32),
            pltpu.VMEM((W, D), jnp.float32),
        ]
        + [pltpu.VMEM((SUB_A, D), jnp.float32) for _ in range(NBUF)]
        + [pltpu.VMEM((SUB_B, D), jnp.float32) for _ in range(NBUF)]
        + [pltpu.SemaphoreType.DMA for _ in range(4 * NBUF)]
    ),
    compiler_params=pltpu.CompilerParams(use_tc_tiling_on_sc=False),
)


def kernel(vector, table, pe):
    vec3 = vector.reshape(NW, NCHUNK, CHUNK)
    idxa = vec3[:, :, :SUB_A]
    idxb = vec3[:, :, SUB_A:]
    out = _embed_call(idxa, idxb, table, pe)
    return out.reshape(B, W, D)


# speedup vs baseline: 1.2458x; 1.0005x over previous
import jax
import jax.numpy as jnp
from jax import lax
from jax.experimental import pallas as pl
from jax.experimental.pallas import tpu as pltpu
from jax.experimental.pallas import tpu_sc as plsc

D = 64
W = 200
B = 4096
NC, NS = 2, 16
NW = NC * NS
TOTAL = B * W
BPW = TOTAL // NW
CHUNK = W
NCHUNK = BPW // CHUNK
SUB_A, SUB_B = 120, 80
LANES = 16
NBUF = 4
PF = 2


def _embed_body(vec3_hbm, table_hbm, pe_hbm, out_hbm,
                idx_all_a, idx_all_b, pe_v,
                ra0, ra1, ra2, ra3, rb0, rb1, rb2, rb3,
                sga0, sga1, sga2, sga3, sgb0, sgb1, sgb2, sgb3,
                ssa0, ssa1, ssa2, ssa3, ssb0, ssb1, ssb2, ssb3):
    ra = [ra0, ra1, ra2, ra3]
    rb = [rb0, rb1, rb2, rb3]
    sga = [sga0, sga1, sga2, sga3]
    sgb = [sgb0, sgb1, sgb2, sgb3]
    ssa = [ssa0, ssa1, ssa2, ssa3]
    ssb = [ssb0, ssb1, ssb2, ssb3]

    wid = lax.axis_index("s") * NC + lax.axis_index("c")
    base0 = wid * BPW
    pltpu.sync_copy(pe_hbm, pe_v)
    pltpu.sync_copy(vec3_hbm.at[wid, :, pl.ds(0, SUB_A)], idx_all_a)
    pltpu.sync_copy(vec3_hbm.at[wid, :, pl.ds(SUB_A, SUB_B)], idx_all_b)

    def start_gather(g, s):
        pltpu.async_copy(table_hbm.at[idx_all_a.at[g]], ra[s], sga[s])
        pltpu.async_copy(table_hbm.at[idx_all_b.at[g]], rb[s], sgb[s])

    def wait_gather(g, s):
        pltpu.make_async_copy(table_hbm.at[idx_all_a.at[g]], ra[s], sga[s]).wait()
        pltpu.make_async_copy(table_hbm.at[idx_all_b.at[g]], rb[s], sgb[s]).wait()

    def start_store(g, s):
        base = base0 + g * CHUNK
        pltpu.async_copy(ra[s], out_hbm.at[pl.ds(base, SUB_A)], ssa[s])
        pltpu.async_copy(rb[s], out_hbm.at[pl.ds(base + SUB_A, SUB_B)], ssb[s])

    def wait_store(g, s):
        base = base0 + g * CHUNK
        pltpu.make_async_copy(ra[s], out_hbm.at[pl.ds(base, SUB_A)], ssa[s]).wait()
        pltpu.make_async_copy(rb[s], out_hbm.at[pl.ds(base + SUB_A, SUB_B)], ssb[s]).wait()

    for s in range(PF):
        start_gather(s, s)

    def outer_body(go, carry):
        for s in range(NBUF):
            g = go * NBUF + s
            sp = (s + PF) % NBUF

            @pl.when(g >= PF)
            def _():
                wait_store(g, sp)

            @pl.when(g + PF < NCHUNK)
            def _():
                start_gather(g + PF, sp)

            wait_gather(g, s)

            def add_a(i, c):
                for j in range(D // LANES):
                    sl = pl.ds(j * LANES, LANES)
                    ra[s][i, sl] = ra[s][i, sl] + pe_v[i, sl]
                return c

            lax.fori_loop(0, SUB_A, add_a, 0)

            def add_b(i, c):
                for j in range(D // LANES):
                    sl = pl.ds(j * LANES, LANES)
                    rb[s][i, sl] = rb[s][i, sl] + pe_v[SUB_A + i, sl]
                return c

            lax.fori_loop(0, SUB_B, add_b, 0)

            start_store(g, s)
        return carry

    lax.fori_loop(0, NCHUNK // NBUF, outer_body, 0)

    for g in range(NCHUNK - NBUF + PF, NCHUNK):
        wait_store(g, g % NBUF)


_embed_call = pl.kernel(
    _embed_body,
    mesh=plsc.VectorSubcoreMesh(core_axis_name="c", subcore_axis_name="s"),
    out_type=jax.ShapeDtypeStruct((TOTAL, D), jnp.float32),
    scratch_types=(
        [
            pltpu.VMEM((NCHUNK, SUB_A), jnp.int32),
            pltpu.VMEM((NCHUNK, SUB_B), jnp.int32),
            pltpu.VMEM((W, D), jnp.float32),
        ]
        + [pltpu.VMEM((SUB_A, D), jnp.float32) for _ in range(NBUF)]
        + [pltpu.VMEM((SUB_B, D), jnp.float32) for _ in range(NBUF)]
        + [pltpu.SemaphoreType.DMA for _ in range(4 * NBUF)]
    ),
    compiler_params=pltpu.CompilerParams(use_tc_tiling_on_sc=False),
)


def kernel(vector, table, pe):
    vec3 = vector.reshape(NW, NCHUNK, CHUNK)
    out = _embed_call(vec3, table, pe)
    return out.reshape(B, W, D)


# speedup vs baseline: 1.2458x; 1.0000x over previous
import jax
import jax.numpy as jnp
from jax import lax
from jax.experimental import pallas as pl
from jax.experimental.pallas import tpu as pltpu
from jax.experimental.pallas import tpu_sc as plsc

D = 64
W = 200
B = 4096
NC, NS = 2, 16
NW = NC * NS
RPW = B // NW
SUB_A, SUB_B = 120, 80
LANES = 16
NBUF = 4
PF = 2


def _embed_body(vec_hbm, table_hbm, pe_hbm, out_hbm,
                idx_all_a, idx_all_b, pe_v,
                ra0, ra1, ra2, ra3, rb0, rb1, rb2, rb3,
                sga0, sga1, sga2, sga3, sgb0, sgb1, sgb2, sgb3,
                ssa0, ssa1, ssa2, ssa3, ssb0, ssb1, ssb2, ssb3):
    ra = [ra0, ra1, ra2, ra3]
    rb = [rb0, rb1, rb2, rb3]
    sga = [sga0, sga1, sga2, sga3]
    sgb = [sgb0, sgb1, sgb2, sgb3]
    ssa = [ssa0, ssa1, ssa2, ssa3]
    ssb = [ssb0, ssb1, ssb2, ssb3]

    wid = lax.axis_index("s") * NC + lax.axis_index("c")
    row0 = wid * RPW
    pltpu.sync_copy(pe_hbm, pe_v)
    pltpu.sync_copy(vec_hbm.at[pl.ds(row0, RPW), pl.ds(0, SUB_A)], idx_all_a)
    pltpu.sync_copy(vec_hbm.at[pl.ds(row0, RPW), pl.ds(SUB_A, SUB_B)], idx_all_b)

    def start_gather(g, s):
        pltpu.async_copy(table_hbm.at[idx_all_a.at[g]], ra[s], sga[s])
        pltpu.async_copy(table_hbm.at[idx_all_b.at[g]], rb[s], sgb[s])

    def wait_gather(g, s):
        pltpu.make_async_copy(table_hbm.at[idx_all_a.at[g]], ra[s], sga[s]).wait()
        pltpu.make_async_copy(table_hbm.at[idx_all_b.at[g]], rb[s], sgb[s]).wait()

    def start_store(g, s):
        row = row0 + g
        pltpu.async_copy(ra[s], out_hbm.at[row, pl.ds(0, SUB_A)], ssa[s])
        pltpu.async_copy(rb[s], out_hbm.at[row, pl.ds(SUB_A, SUB_B)], ssb[s])

    def wait_store(g, s):
        row = row0 + g
        pltpu.make_async_copy(ra[s], out_hbm.at[row, pl.ds(0, SUB_A)], ssa[s]).wait()
        pltpu.make_async_copy(rb[s], out_hbm.at[row, pl.ds(SUB_A, SUB_B)], ssb[s]).wait()

    def process(g, s):
        wait_gather(g, s)

        def add_a(i, c):
            for j in range(D // LANES):
                sl = pl.ds(j * LANES, LANES)
                ra[s][i, sl] = ra[s][i, sl] + pe_v[i, sl]
            return c

        lax.fori_loop(0, SUB_A, add_a, 0)

        def add_b(i, c):
            for j in range(D // LANES):
                sl = pl.ds(j * LANES, LANES)
                rb[s][i, sl] = rb[s][i, sl] + pe_v[SUB_A + i, sl]
            return c

        lax.fori_loop(0, SUB_B, add_b, 0)

        start_store(g, s)

    for s in range(PF):
        start_gather(s, s)
    for g in range(PF):
        start_gather(g + PF, (g + PF) % NBUF)
        process(g, g)

    def outer_body(go, carry):
        for s in range(NBUF):
            g = PF + go * NBUF + s
            cs = (PF + s) % NBUF
            ps = s % NBUF
            wait_store(g - PF, ps)
            start_gather(g + PF, ps)
            process(g, cs)
        return carry

    lax.fori_loop(0, (RPW - 2 * PF) // NBUF, outer_body, 0)

    for g in range(RPW - PF, RPW):
        wait_store(g - PF, (g - PF) % NBUF)
        process(g, g % NBUF)

    for g in range(RPW - PF, RPW):
        wait_store(g, g % NBUF)


_embed_call = pl.kernel(
    _embed_body,
    mesh=plsc.VectorSubcoreMesh(core_axis_name="c", subcore_axis_name="s"),
    out_type=jax.ShapeDtypeStruct((B, W, D), jnp.float32),
    scratch_types=(
        [
            pltpu.VMEM((RPW, SUB_A), jnp.int32),
            pltpu.VMEM((RPW, SUB_B), jnp.int32),
            pltpu.VMEM((W, D), jnp.float32),
        ]
        + [pltpu.VMEM((SUB_A, D), jnp.float32) for _ in range(NBUF)]
        + [pltpu.VMEM((SUB_B, D), jnp.float32) for _ in range(NBUF)]
        + [pltpu.SemaphoreType.DMA for _ in range(4 * NBUF)]
    ),
    compiler_params=pltpu.CompilerParams(use_tc_tiling_on_sc=False),
)


def kernel(vector, table, pe):
    return _embed_call(vector, table, pe)
